# baseline (speedup 1.0000x reference)
import functools

import jax
import jax.numpy as jnp
from jax import lax
from jax.experimental import pallas as pl
from jax.experimental.pallas import tpu as pltpu
from jax.experimental.pallas import tpu_sc as plsc

N = 100000
E = 1600000
G = 128
D = 32
NCLS = 10

NC = 2
NS = 16
L = 16
K = 128

HALF = N // NC
NBUF = 4
EP = 100352
E2 = EP * NS
NCHUNK = EP // K
ACC_ROWS = 51200
TRASH = HALF
ZCH = ACC_ROWS // NS // K

BN = 2048
NBLK = (N + BN - 1) // BN


def _sc_agg_body(want_deg, *refs):
    if want_deg:
        (g_hbm, src_hbm, dst_hbm, agg_out, deg_out,
         src_v, dst_v, idx_v, rows_v, zrow_v, ones_v, acc_sh,
         sd_sem, g_sem, s_sem, zvec_v, dacc_sh, o_sem) = refs
    else:
        (g_hbm, src_hbm, dst_hbm, agg_out,
         src_v, dst_v, idx_v, rows_v, zrow_v, ones_v, acc_sh,
         sd_sem, g_sem, s_sem) = refs
        o_sem = None

    c = lax.axis_index("c")
    s = lax.axis_index("s")
    base_node = c * HALF

    @pl.loop(0, K)
    def _fill(i):
        zrow_v[i, pl.ds(0, L)] = jnp.zeros((L,), jnp.float32)
        zrow_v[i, pl.ds(L, L)] = jnp.zeros((L,), jnp.float32)
    if want_deg:
        for j in range(K // L):
            ones_v[pl.ds(j * L, L)] = jnp.ones((L,), jnp.float32)
            zvec_v[pl.ds(j * L, L)] = jnp.zeros((L,), jnp.float32)

    @pl.loop(0, ZCH)
    def _zero(i):
        r0 = (s * ZCH + i) * K
        pltpu.sync_copy(zrow_v, acc_sh.at[pl.ds(r0, K)])
        if want_deg:
            pltpu.sync_copy(zvec_v, dacc_sh.at[pl.ds(r0, K)])
    plsc.subcore_barrier()

    ebase = s * EP

    def _fire_loads(tt, b):
        off = ebase + tt * K
        pltpu.async_copy(src_hbm.at[pl.ds(off, K)], src_v.at[b], sd_sem.at[b])
        pltpu.async_copy(dst_hbm.at[pl.ds(off, K)], dst_v.at[b], sd_sem.at[b])

    def _wait_loads(b):
        pltpu.make_async_copy(src_hbm.at[pl.ds(0, K)], src_v.at[b],
                              sd_sem.at[b]).wait()
        pltpu.make_async_copy(dst_hbm.at[pl.ds(0, K)], dst_v.at[b],
                              sd_sem.at[b]).wait()

    for b in range(NBUF):
        _fire_loads(b, b)

    @pl.loop(0, NCHUNK, step=NBUF)
    def _edges(t):
        for b in range(NBUF):
            tt = t + b
            _wait_loads(b)

            @pl.when(tt >= NBUF)
            def _():
                pltpu.make_async_copy(rows_v.at[b], acc_sh.at[idx_v.at[b]],
                                      s_sem.at[b]).wait()
                if want_deg:
                    pltpu.make_async_copy(ones_v, dacc_sh.at[idx_v.at[b]],
                                          o_sem.at[b]).wait()
            pltpu.async_copy(g_hbm.at[src_v.at[b]], rows_v.at[b], g_sem.at[b])
        for b in range(NBUF):
            tt = t + b
            for j in range(K // L):
                d = dst_v[b, pl.ds(j * L, L)]
                loc = d - base_node
                ok = (loc >= 0) & (loc < HALF)
                idx_v[b, pl.ds(j * L, L)] = jnp.where(
                    ok, loc, TRASH + s * 64 + j * 8)
            pltpu.make_async_copy(g_hbm.at[src_v.at[b]], rows_v.at[b],
                                  g_sem.at[b]).wait()

            @pl.when(tt + NBUF < NCHUNK)
            def _():
                _fire_loads(tt + NBUF, b)
            pltpu.async_copy(rows_v.at[b], acc_sh.at[idx_v.at[b]],
                             s_sem.at[b], add=True)
            if want_deg:
                pltpu.async_copy(ones_v, dacc_sh.at[idx_v.at[b]],
                                 o_sem.at[b], add=True)

    for b in range(NBUF):
        pltpu.make_async_copy(rows_v.at[b], acc_sh.at[idx_v.at[b]],
                              s_sem.at[b]).wait()
        if want_deg:
            pltpu.make_async_copy(ones_v, dacc_sh.at[idx_v.at[b]],
                                  o_sem.at[b]).wait()
    plsc.subcore_barrier()

    rpt = 5000

    @pl.when(s < 10)
    def _():
        r0 = s * rpt
        pltpu.sync_copy(acc_sh.at[pl.ds(r0, rpt)],
                        agg_out.at[pl.ds(base_node + r0, rpt)])
        if want_deg:
            pltpu.sync_copy(dacc_sh.at[pl.ds(r0, rpt)],
                            deg_out.at[pl.ds(base_node + r0, rpt)])


def _make_sc_agg(want_deg):
    out_type = [jax.ShapeDtypeStruct((N, D), jnp.float32)]
    scratch = [
        pltpu.VMEM((NBUF, K), jnp.int32),
        pltpu.VMEM((NBUF, K), jnp.int32),
        pltpu.VMEM((NBUF, K), jnp.int32),
        pltpu.VMEM((NBUF, K, D), jnp.float32),
        pltpu.VMEM((K, D), jnp.float32),
        pltpu.VMEM((K,), jnp.float32),
        pltpu.VMEM_SHARED((ACC_ROWS, D), jnp.float32),
        pltpu.SemaphoreType.DMA((NBUF,)),
        pltpu.SemaphoreType.DMA((NBUF,)),
        pltpu.SemaphoreType.DMA((NBUF,)),
    ]
    if want_deg:
        out_type = out_type + [jax.ShapeDtypeStruct((N,), jnp.float32)]
        scratch = scratch + [pltpu.VMEM((K,), jnp.float32),
                             pltpu.VMEM_SHARED((ACC_ROWS,), jnp.float32),
                             pltpu.SemaphoreType.DMA((NBUF,))]
    return pl.kernel(
        functools.partial(_sc_agg_body, want_deg),
        out_type=tuple(out_type),
        mesh=plsc.VectorSubcoreMesh(core_axis_name="c", subcore_axis_name="s"),
        scratch_types=scratch,
        compiler_params=pltpu.CompilerParams(use_tc_tiling_on_sc=False),
    )


def _dot(a, b):
    return jax.lax.dot_general(a, b, (((1,), (0,)), ((), ())),
                               preferred_element_type=jnp.float32)


def _dotT(a, b):
    return jax.lax.dot_general(a, b, (((0,), (0,)), ((), ())),
                               preferred_element_type=jnp.float32)


def _embed_body(xs_ref, xc_ref, semb_ref, cemb_ref, wl1t_ref, wr1t_ref,
                bl1_ref, g1_ref, r1_ref):
    oh_s = (xs_ref[...] ==
            lax.broadcasted_iota(jnp.int32, (BN, 16), 1)).astype(jnp.float32)
    oh_c = (xc_ref[...] ==
            lax.broadcasted_iota(jnp.int32, (BN, 16), 1)).astype(jnp.float32)
    h0 = jnp.concatenate(
        [_dot(oh_s, semb_ref[...]), _dot(oh_c, cemb_ref[...])], axis=1)
    g1_ref[...] = _dot(h0, wl1t_ref[...])
    r1_ref[...] = _dot(h0, wr1t_ref[...]) + bl1_ref[...]


def _mid_body(agg_ref, deg_ref, r_ref, wlt_ref, wrt_ref, bl_ref,
              g2_ref, r2_ref):
    inv = 1.0 / jnp.clip(deg_ref[...], 1.0, None)
    h = jnp.maximum(agg_ref[...] * inv + r_ref[...], 0.0)
    g2_ref[...] = _dot(h, wlt_ref[...])
    r2_ref[...] = _dot(h, wrt_ref[...]) + bl_ref[...]


def _tail_body(agg_ref, deg_ref, r_ref, batch_ref, fcwt_ref, fcb_ref,
               out_ref, sums_ref, cnts_ref):
    i = pl.program_id(0)

    @pl.when(i == 0)
    def _():
        sums_ref[...] = jnp.zeros_like(sums_ref)
        cnts_ref[...] = jnp.zeros_like(cnts_ref)

    valid = (i * BN + lax.broadcasted_iota(jnp.int32, (BN, 1), 0)) < N
    inv = 1.0 / jnp.clip(deg_ref[...], 1.0, None)
    h = jnp.maximum(agg_ref[...] * inv + r_ref[...], 0.0)
    h = jnp.where(valid, h, 0.0)
    b = jnp.where(valid, batch_ref[...], -1)
    oh = (b == lax.broadcasted_iota(jnp.int32, (BN, G), 1)).astype(jnp.float32)
    sums_ref[...] += _dotT(oh, h)
    cnts_ref[...] += _dotT(oh, jnp.ones((BN, 1), jnp.float32))

    @pl.when(i == NBLK - 1)
    def _():
        pooled = sums_ref[...] * (1.0 / jnp.clip(cnts_ref[...], 1.0, None))
        out_ref[...] = _dot(pooled, fcwt_ref[...]) + fcb_ref[...]


def _node_spec(width=1):
    return pl.BlockSpec((BN, width), lambda i: (i, 0))


def _full_spec(shape):
    nd = len(shape)
    return pl.BlockSpec(shape, lambda i: (0,) * nd)


def kernel(x, edge_index, batch, shape_emb, color_emb,
           Wl1, bl1, Wr1, Wl2, bl2, Wr2, fcW, fcb):
    xs = x[:, 0:1].astype(jnp.int32)
    xc = x[:, 1:2].astype(jnp.int32)
    src = edge_index[0].astype(jnp.int32)
    dst = edge_index[1].astype(jnp.int32)
    pad = E2 - E
    src_p = jnp.concatenate([src, jnp.zeros((pad,), jnp.int32)])
    dst_p = jnp.concatenate([dst, jnp.full((pad,), N, jnp.int32)])

    g1, r1 = pl.pallas_call(
        _embed_body,
        grid=(NBLK,),
        in_specs=[_node_spec(), _node_spec(), _full_spec((16, 16)),
                  _full_spec((16, 16)), _full_spec((D, D)),
                  _full_spec((D, D)), _full_spec((1, D))],
        out_specs=[_node_spec(D), _node_spec(D)],
        out_shape=[jax.ShapeDtypeStruct((N, D), jnp.float32),
                   jax.ShapeDtypeStruct((N, D), jnp.float32)],
    )(xs, xc, shape_emb, color_emb, Wl1.T, Wr1.T, bl1[None, :])

    agg1, deg = _make_sc_agg(True)(g1, src_p, dst_p)
    deg2 = deg[:, None]

    g2, r2 = pl.pallas_call(
        _mid_body,
        grid=(NBLK,),
        in_specs=[_node_spec(D), _node_spec(), _node_spec(D),
                  _full_spec((D, D)), _full_spec((D, D)), _full_spec((1, D))],
        out_specs=[_node_spec(D), _node_spec(D)],
        out_shape=[jax.ShapeDtypeStruct((N, D), jnp.float32),
                   jax.ShapeDtypeStruct((N, D), jnp.float32)],
    )(agg1, deg2, r1, Wl2.T, Wr2.T, bl2[None, :])

    (agg2,) = _make_sc_agg(False)(g2, src_p, dst_p)

    fcwt_pad = jnp.zeros((D, 16), jnp.float32).at[:, :NCLS].set(fcW.T)
    fcb_pad = jnp.zeros((1, 16), jnp.float32).at[0, :NCLS].set(fcb)
    out16 = pl.pallas_call(
        _tail_body,
        grid=(NBLK,),
        in_specs=[_node_spec(D), _node_spec(), _node_spec(D), _node_spec(),
                  _full_spec((D, 16)), _full_spec((1, 16))],
        out_specs=_full_spec((G, 16)),
        out_shape=jax.ShapeDtypeStruct((G, 16), jnp.float32),
        scratch_shapes=[pltpu.VMEM((G, D), jnp.float32),
                        pltpu.VMEM((G, 1), jnp.float32)],
    )(agg2, deg2, r2, batch.astype(jnp.int32)[:, None], fcwt_pad, fcb_pad)
    return out16[:, :NCLS]


# speedup vs baseline: 10.9059x; 10.9059x over previous
import functools

import jax
import jax.numpy as jnp
from jax import lax
from jax.experimental import pallas as pl
from jax.experimental.pallas import tpu as pltpu
from jax.experimental.pallas import tpu_sc as plsc

N = 100000
E = 1600000
G = 128
D = 32
NCLS = 10

NC = 2
NS = 16
L = 16
K = 128

HALF = N // NC
NBUF = 4
EP = 100352
E2 = EP * NS
NCHUNK = EP // K
ACC_ROWS = 51200
TRASH = HALF
ZCH = ACC_ROWS // NS // K

BN = 2048
NBLK = (N + BN - 1) // BN


FL = 512
FLC = FL // K
EPW = E2 // (NC * NS)
PCH = EPW // K
CBUF = 2 * FL + K + L
NSPAN = 2 * NC * NS
SPAN_CAP = EPW + FL
PTOT = NSPAN * SPAN_CAP


def _sc_part_body(src_hbm, dst_hbm, psrc_out, ploc_out, pcnt_out,
                  src_v, dst_v, csl, cll, csh, clh, cntv_v,
                  sd_sem, fsem_lo, fsem_hi):
    c = lax.axis_index("c")
    s = lax.axis_index("s")
    w = c * NS + s
    ebase = w * EPW
    trash = TRASH + w * 4
    base_lo = w * SPAN_CAP
    base_hi = (NSPAN // 2 + w) * SPAN_CAP

    @pl.loop(0, CBUF // L)
    def _init(i):
        z = jnp.zeros((L,), jnp.int32)
        csl[pl.ds(i * L, L)] = z
        csh[pl.ds(i * L, L)] = z
        cll[pl.ds(i * L, L)] = z + trash
        clh[pl.ds(i * L, L)] = z + trash

    pltpu.async_copy(csl.at[pl.ds(0, FL)],
                     psrc_out.at[pl.ds(base_lo, FL)], fsem_lo)
    pltpu.async_copy(cll.at[pl.ds(0, FL)],
                     ploc_out.at[pl.ds(base_lo, FL)], fsem_lo)
    pltpu.async_copy(csh.at[pl.ds(0, FL)],
                     psrc_out.at[pl.ds(base_hi, FL)], fsem_hi)
    pltpu.async_copy(clh.at[pl.ds(0, FL)],
                     ploc_out.at[pl.ds(base_hi, FL)], fsem_hi)

    def _fire_loads(tt, b):
        off = ebase + tt * K
        pltpu.async_copy(src_hbm.at[pl.ds(off, K)], src_v.at[b], sd_sem.at[b])
        pltpu.async_copy(dst_hbm.at[pl.ds(off, K)], dst_v.at[b], sd_sem.at[b])

    def _wait_loads(b):
        pltpu.make_async_copy(src_hbm.at[pl.ds(0, K)], src_v.at[b],
                              sd_sem.at[b]).wait()
        pltpu.make_async_copy(dst_hbm.at[pl.ds(0, K)], dst_v.at[b],
                              sd_sem.at[b]).wait()

    def _wait_flush(cs, cl, fsem):
        pltpu.make_async_copy(cs.at[pl.ds(0, FL)],
                              psrc_out.at[pl.ds(0, FL)], fsem).wait()
        pltpu.make_async_copy(cl.at[pl.ds(0, FL)],
                              ploc_out.at[pl.ds(0, FL)], fsem).wait()

    def _fire_flush(cs, cl, fsem, span_base, fl, region):
        off = span_base + fl * FL
        pltpu.async_copy(cs.at[pl.ds(region, FL)],
                         psrc_out.at[pl.ds(off, FL)], fsem)
        pltpu.async_copy(cl.at[pl.ds(region, FL)],
                         ploc_out.at[pl.ds(off, FL)], fsem)

    def _flush_step(cnt, fl, cs, cl, fsem, span_base):
        par = lax.rem(fl, 2)
        p0 = (cnt >= FL) & (par == 0)
        p1 = (cnt >= 2 * FL) & (par == 1)

        @pl.when(p0)
        def _():
            _wait_flush(cs, cl, fsem)
            _fire_flush(cs, cl, fsem, span_base, fl, 0)

        @pl.when(p1)
        def _():
            _wait_flush(cs, cl, fsem)
            _fire_flush(cs, cl, fsem, span_base, fl, FL)
            for i in range(K // L):
                cs[pl.ds(i * L, L)] = cs[pl.ds(2 * FL + i * L, L)]
                cl[pl.ds(i * L, L)] = cl[pl.ds(2 * FL + i * L, L)]
        cnt = jnp.where(p1, cnt - 2 * FL, cnt)
        fl = fl + jnp.where(p0 | p1, 1, 0).astype(jnp.int32)
        return cnt, fl

    for b in range(NBUF):
        _fire_loads(b, b)

    zero = jnp.int32(0)

    @pl.loop(0, PCH, step=NBUF, init_carry=(zero, zero, zero, zero))
    def _scan(t, carry):
        cnt_lo, fl_lo, cnt_hi, fl_hi = carry
        for b in range(NBUF):
            tt = t + b
            _wait_loads(b)
            for j in range(K // L):
                sv = src_v[b, pl.ds(j * L, L)]
                dv = dst_v[b, pl.ds(j * L, L)]
                m_lo = dv < HALF
                m_hi = (dv >= HALF) & (dv < N)
                cum_lo = plsc.cumsum(m_lo.astype(jnp.int32))
                pos_lo = cnt_lo + cum_lo - 1
                plsc.store_scatter(csl, [pos_lo], sv, mask=m_lo)
                plsc.store_scatter(cll, [pos_lo], dv, mask=m_lo)
                cnt_lo = cnt_lo + cum_lo[L - 1]
                cum_hi = plsc.cumsum(m_hi.astype(jnp.int32))
                pos_hi = cnt_hi + cum_hi - 1
                plsc.store_scatter(csh, [pos_hi], sv, mask=m_hi)
                plsc.store_scatter(clh, [pos_hi], dv - HALF, mask=m_hi)
                cnt_hi = cnt_hi + cum_hi[L - 1]

            @pl.when(tt + NBUF < PCH)
            def _():
                _fire_loads(tt + NBUF, b)
            cnt_lo, fl_lo = _flush_step(cnt_lo, fl_lo, csl, cll,
                                        fsem_lo, base_lo)
            cnt_hi, fl_hi = _flush_step(cnt_hi, fl_hi, csh, clh,
                                        fsem_hi, base_hi)
        return cnt_lo, fl_lo, cnt_hi, fl_hi

    cnt_lo, fl_lo, cnt_hi, fl_hi = _scan
    full = jnp.ones((L,), jnp.bool_)
    tr_src = jnp.zeros((L,), jnp.int32)
    tr_loc = tr_src + trash

    def _drain(cnt, fl, cs, cl, fsem, span_base):
        pad = lax.rem(FL - lax.rem(cnt, FL), FL)
        pad16 = (pad + L - 1) // L

        iota16 = lax.broadcasted_iota(jnp.int32, (L,), 0)

        @pl.loop(0, pad16)
        def _pad(i):
            pos = cnt + i * L + iota16
            plsc.store_scatter(cs, [pos], tr_src, mask=full)
            plsc.store_scatter(cl, [pos], tr_loc, mask=full)
        cnt2 = cnt + pad
        pend = cnt2 // FL - lax.rem(fl, 2)

        @pl.when(pend >= 1)
        def _():
            _wait_flush(cs, cl, fsem)
            _fire_flush(cs, cl, fsem, span_base, fl, lax.rem(fl, 2) * FL)
        fl1 = fl + (pend >= 1).astype(jnp.int32)

        @pl.when(pend >= 2)
        def _():
            _wait_flush(cs, cl, fsem)
            _fire_flush(cs, cl, fsem, span_base, fl1, lax.rem(fl1, 2) * FL)
        fl2 = fl1 + (pend >= 2).astype(jnp.int32)
        _wait_flush(cs, cl, fsem)
        return fl2

    nfl_lo = _drain(cnt_lo, fl_lo, csl, cll, fsem_lo, base_lo)
    nfl_hi = _drain(cnt_hi, fl_hi, csh, clh, fsem_hi, base_hi)

    lane0 = lax.broadcasted_iota(jnp.int32, (L,), 0) == 0
    cntv_v[...] = jnp.where(lane0, nfl_lo, 0)
    pltpu.sync_copy(cntv_v, pcnt_out.at[pl.ds(w * L, L)])
    cntv_v[...] = jnp.where(lane0, nfl_hi, 0)
    pltpu.sync_copy(cntv_v, pcnt_out.at[pl.ds((NSPAN // 2 + w) * L, L)])


def _make_sc_part():
    return pl.kernel(
        _sc_part_body,
        out_type=(jax.ShapeDtypeStruct((PTOT,), jnp.int32),
                  jax.ShapeDtypeStruct((PTOT,), jnp.int32),
                  jax.ShapeDtypeStruct((NSPAN * L,), jnp.int32)),
        mesh=plsc.VectorSubcoreMesh(core_axis_name="c", subcore_axis_name="s"),
        scratch_types=[
            pltpu.VMEM((NBUF, K), jnp.int32),
            pltpu.VMEM((NBUF, K), jnp.int32),
            pltpu.VMEM((CBUF,), jnp.int32),
            pltpu.VMEM((CBUF,), jnp.int32),
            pltpu.VMEM((CBUF,), jnp.int32),
            pltpu.VMEM((CBUF,), jnp.int32),
            pltpu.VMEM((L,), jnp.int32),
            pltpu.SemaphoreType.DMA((NBUF,)),
            pltpu.SemaphoreType.DMA,
            pltpu.SemaphoreType.DMA,
        ],
        compiler_params=pltpu.CompilerParams(use_tc_tiling_on_sc=False,
                                             needs_layout_passes=False),
    )


def _sc_agg_body(want_deg, *refs):
    if want_deg:
        (g_hbm, psrc, ploc, pcnt, agg_out, deg_out,
         src_v, loc_v, idx_v, rows_v, zrow_v, ones_v, acc_sh, cnt_v,
         sd_sem, g_sem, s_sem, zvec_v, dacc_sh, o_sem) = refs
    else:
        (g_hbm, psrc, ploc, pcnt, agg_out,
         src_v, loc_v, idx_v, rows_v, zrow_v, ones_v, acc_sh, cnt_v,
         sd_sem, g_sem, s_sem) = refs
        o_sem = None

    c = lax.axis_index("c")
    s = lax.axis_index("s")
    base_node = c * HALF

    @pl.loop(0, K)
    def _fill(i):
        zrow_v[i, pl.ds(0, L)] = jnp.zeros((L,), jnp.float32)
        zrow_v[i, pl.ds(L, L)] = jnp.zeros((L,), jnp.float32)
    if want_deg:
        for j in range(K // L):
            ones_v[pl.ds(j * L, L)] = jnp.ones((L,), jnp.float32)
            zvec_v[pl.ds(j * L, L)] = jnp.zeros((L,), jnp.float32)

    @pl.loop(0, ZCH)
    def _zero(i):
        r0 = (s * ZCH + i) * K
        pltpu.sync_copy(zrow_v, acc_sh.at[pl.ds(r0, K)])
        if want_deg:
            pltpu.sync_copy(zvec_v, dacc_sh.at[pl.ds(r0, K)])
    plsc.subcore_barrier()

    for which in range(2):
        sp = c * (NSPAN // 2) + 2 * s + which
        base = sp * SPAN_CAP
        pltpu.sync_copy(pcnt.at[pl.ds(sp * L, L)], cnt_v)
        nch = cnt_v[...][0] * FLC

        def _fire(tt, b):
            off = base + tt * K
            pltpu.async_copy(psrc.at[pl.ds(off, K)], src_v.at[b],
                             sd_sem.at[b])
            pltpu.async_copy(ploc.at[pl.ds(off, K)], loc_v.at[b],
                             sd_sem.at[b])

        for b in range(NBUF):
            @pl.when(b < nch)
            def _():
                _fire(b, b)

        @pl.loop(0, nch, step=NBUF)
        def _edges(t):
            for b in range(NBUF):
                tt = t + b
                pltpu.make_async_copy(psrc.at[pl.ds(0, K)], src_v.at[b],
                                      sd_sem.at[b]).wait()
                pltpu.make_async_copy(ploc.at[pl.ds(0, K)], loc_v.at[b],
                                      sd_sem.at[b]).wait()

                @pl.when(tt >= NBUF)
                def _():
                    pltpu.make_async_copy(rows_v.at[b],
                                          acc_sh.at[idx_v.at[b]],
                                          s_sem.at[b]).wait()
                    if want_deg:
                        pltpu.make_async_copy(ones_v,
                                              dacc_sh.at[idx_v.at[b]],
                                              o_sem.at[b]).wait()
                pltpu.async_copy(g_hbm.at[src_v.at[b]], rows_v.at[b],
                                 g_sem.at[b])
            for b in range(NBUF):
                tt = t + b
                for j in range(K // L):
                    idx_v[b, pl.ds(j * L, L)] = loc_v[b, pl.ds(j * L, L)]
                pltpu.make_async_copy(g_hbm.at[src_v.at[b]], rows_v.at[b],
                                      g_sem.at[b]).wait()

                @pl.when(tt + NBUF < nch)
                def _():
                    _fire(tt + NBUF, b)
                pltpu.async_copy(rows_v.at[b], acc_sh.at[idx_v.at[b]],
                                 s_sem.at[b], add=True)
                if want_deg:
                    pltpu.async_copy(ones_v, dacc_sh.at[idx_v.at[b]],
                                     o_sem.at[b], add=True)

        for b in range(NBUF):
            @pl.when(b < nch)
            def _():
                pltpu.make_async_copy(rows_v.at[b], acc_sh.at[idx_v.at[b]],
                                      s_sem.at[b]).wait()
                if want_deg:
                    pltpu.make_async_copy(ones_v, dacc_sh.at[idx_v.at[b]],
                                          o_sem.at[b]).wait()
    plsc.subcore_barrier()

    rpt = 5000

    @pl.when(s < 10)
    def _():
        r0 = s * rpt
        pltpu.sync_copy(acc_sh.at[pl.ds(r0, rpt)],
                        agg_out.at[pl.ds(base_node + r0, rpt)])
        if want_deg:
            pltpu.sync_copy(dacc_sh.at[pl.ds(r0, rpt)],
                            deg_out.at[pl.ds(base_node + r0, rpt)])


def _make_sc_agg(want_deg):
    out_type = [jax.ShapeDtypeStruct((N, D), jnp.float32)]
    scratch = [
        pltpu.VMEM((NBUF, K), jnp.int32),
        pltpu.VMEM((NBUF, K), jnp.int32),
        pltpu.VMEM((NBUF, K), jnp.int32),
        pltpu.VMEM((NBUF, K, D), jnp.float32),
        pltpu.VMEM((K, D), jnp.float32),
        pltpu.VMEM((K,), jnp.float32),
        pltpu.VMEM_SHARED((ACC_ROWS, D), jnp.float32),
        pltpu.VMEM((L,), jnp.int32),
        pltpu.SemaphoreType.DMA((NBUF,)),
        pltpu.SemaphoreType.DMA((NBUF,)),
        pltpu.SemaphoreType.DMA((NBUF,)),
    ]
    if want_deg:
        out_type = out_type + [jax.ShapeDtypeStruct((N,), jnp.float32)]
        scratch = scratch + [pltpu.VMEM((K,), jnp.float32),
                             pltpu.VMEM_SHARED((ACC_ROWS,), jnp.float32),
                             pltpu.SemaphoreType.DMA((NBUF,))]
    return pl.kernel(
        functools.partial(_sc_agg_body, want_deg),
        out_type=tuple(out_type),
        mesh=plsc.VectorSubcoreMesh(core_axis_name="c", subcore_axis_name="s"),
        scratch_types=scratch,
        compiler_params=pltpu.CompilerParams(use_tc_tiling_on_sc=False),
    )


def _dot(a, b):
    return jax.lax.dot_general(a, b, (((1,), (0,)), ((), ())),
                               preferred_element_type=jnp.float32)


def _dotT(a, b):
    return jax.lax.dot_general(a, b, (((0,), (0,)), ((), ())),
                               preferred_element_type=jnp.float32)


def _embed_body(xs_ref, xc_ref, semb_ref, cemb_ref, wl1t_ref, wr1t_ref,
                bl1_ref, g1_ref, r1_ref):
    oh_s = (xs_ref[...] ==
            lax.broadcasted_iota(jnp.int32, (BN, 16), 1)).astype(jnp.float32)
    oh_c = (xc_ref[...] ==
            lax.broadcasted_iota(jnp.int32, (BN, 16), 1)).astype(jnp.float32)
    h0 = jnp.concatenate(
        [_dot(oh_s, semb_ref[...]), _dot(oh_c, cemb_ref[...])], axis=1)
    g1_ref[...] = _dot(h0, wl1t_ref[...])
    r1_ref[...] = _dot(h0, wr1t_ref[...]) + bl1_ref[...]


def _mid_body(agg_ref, deg_ref, r_ref, wlt_ref, wrt_ref, bl_ref,
              g2_ref, r2_ref):
    inv = 1.0 / jnp.clip(deg_ref[...], 1.0, None)
    h = jnp.maximum(agg_ref[...] * inv + r_ref[...], 0.0)
    g2_ref[...] = _dot(h, wlt_ref[...])
    r2_ref[...] = _dot(h, wrt_ref[...]) + bl_ref[...]


def _tail_body(agg_ref, deg_ref, r_ref, batch_ref, fcwt_ref, fcb_ref,
               out_ref, sums_ref, cnts_ref):
    i = pl.program_id(0)

    @pl.when(i == 0)
    def _():
        sums_ref[...] = jnp.zeros_like(sums_ref)
        cnts_ref[...] = jnp.zeros_like(cnts_ref)

    valid = (i * BN + lax.broadcasted_iota(jnp.int32, (BN, 1), 0)) < N
    inv = 1.0 / jnp.clip(deg_ref[...], 1.0, None)
    h = jnp.maximum(agg_ref[...] * inv + r_ref[...], 0.0)
    h = jnp.where(valid, h, 0.0)
    b = jnp.where(valid, batch_ref[...], -1)
    oh = (b == lax.broadcasted_iota(jnp.int32, (BN, G), 1)).astype(jnp.float32)
    sums_ref[...] += _dotT(oh, h)
    cnts_ref[...] += _dotT(oh, jnp.ones((BN, 1), jnp.float32))

    @pl.when(i == NBLK - 1)
    def _():
        pooled = sums_ref[...] * (1.0 / jnp.clip(cnts_ref[...], 1.0, None))
        out_ref[...] = _dot(pooled, fcwt_ref[...]) + fcb_ref[...]


def _node_spec(width=1):
    return pl.BlockSpec((BN, width), lambda i: (i, 0))


def _full_spec(shape):
    nd = len(shape)
    return pl.BlockSpec(shape, lambda i: (0,) * nd)


def kernel(x, edge_index, batch, shape_emb, color_emb,
           Wl1, bl1, Wr1, Wl2, bl2, Wr2, fcW, fcb):
    xs = x[:, 0:1].astype(jnp.int32)
    xc = x[:, 1:2].astype(jnp.int32)
    src = edge_index[0].astype(jnp.int32)
    dst = edge_index[1].astype(jnp.int32)
    pad = E2 - E
    src_p = jnp.concatenate([src, jnp.zeros((pad,), jnp.int32)])
    dst_p = jnp.concatenate([dst, jnp.full((pad,), N, jnp.int32)])

    g1, r1 = pl.pallas_call(
        _embed_body,
        grid=(NBLK,),
        in_specs=[_node_spec(), _node_spec(), _full_spec((16, 16)),
                  _full_spec((16, 16)), _full_spec((D, D)),
                  _full_spec((D, D)), _full_spec((1, D))],
        out_specs=[_node_spec(D), _node_spec(D)],
        out_shape=[jax.ShapeDtypeStruct((N, D), jnp.float32),
                   jax.ShapeDtypeStruct((N, D), jnp.float32)],
    )(xs, xc, shape_emb, color_emb, Wl1.T, Wr1.T, bl1[None, :])

    psrc, ploc, pcnt = _make_sc_part()(src_p, dst_p)

    agg1, deg = _make_sc_agg(True)(g1, psrc, ploc, pcnt)
    deg2 = deg[:, None]

    g2, r2 = pl.pallas_call(
        _mid_body,
        grid=(NBLK,),
        in_specs=[_node_spec(D), _node_spec(), _node_spec(D),
                  _full_spec((D, D)), _full_spec((D, D)), _full_spec((1, D))],
        out_specs=[_node_spec(D), _node_spec(D)],
        out_shape=[jax.ShapeDtypeStruct((N, D), jnp.float32),
                   jax.ShapeDtypeStruct((N, D), jnp.float32)],
    )(agg1, deg2, r1, Wl2.T, Wr2.T, bl2[None, :])

    (agg2,) = _make_sc_agg(False)(g2, psrc, ploc, pcnt)

    fcwt_pad = jnp.zeros((D, 16), jnp.float32).at[:, :NCLS].set(fcW.T)
    fcb_pad = jnp.zeros((1, 16), jnp.float32).at[0, :NCLS].set(fcb)
    out16 = pl.pallas_call(
        _tail_body,
        grid=(NBLK,),
        in_specs=[_node_spec(D), _node_spec(), _node_spec(D), _node_spec(),
                  _full_spec((D, 16)), _full_spec((1, 16))],
        out_specs=_full_spec((G, 16)),
        out_shape=jax.ShapeDtypeStruct((G, 16), jnp.float32),
        scratch_shapes=[pltpu.VMEM((G, D), jnp.float32),
                        pltpu.VMEM((G, 1), jnp.float32)],
    )(agg2, deg2, r2, batch.astype(jnp.int32)[:, None], fcwt_pad, fcb_pad)
    return out16[:, :NCLS]
